# baseline (speedup 1.0000x reference)
import functools

import jax
import jax.numpy as jnp
from jax import lax
from jax.experimental import pallas as pl
from jax.experimental.pallas import tpu as pltpu
from jax.experimental.pallas import tpu_sc as plsc

_EPS = 1e-5
_LANES = 16


def _ps_table_body(pos_ref, seg_ref, out_ref, *, S, NSEG):
    for s in range(NSEG):
        out_ref[pl.ds(s * S, S), :] = pos_ref[...] + seg_ref[pl.ds(s, 1), :]


def _build_ps_table(pos_embed, seg_embed, S):
    NSEG, EMB = seg_embed.shape
    return pl.pallas_call(
        functools.partial(_ps_table_body, S=S, NSEG=NSEG),
        out_shape=jax.ShapeDtypeStruct((NSEG * S, EMB), jnp.float32),
    )(pos_embed[:S], seg_embed)


def _lane_sum(v):
    base = lax.iota(jnp.int32, _LANES)
    for sh in (8, 4, 2, 1):
        idx = jnp.bitwise_xor(base, sh)
        v = v + v.at[idx].get(mode="promise_in_bounds", unique_indices=True)
    return v


def _rsqrt(v):
    vi = lax.bitcast_convert_type(v, jnp.int32)
    yi = jnp.int32(0x5F3759DF) - lax.shift_right_logical(vi, 1)
    y = lax.bitcast_convert_type(yi, jnp.float32)
    for _ in range(3):
        y = y * (1.5 - 0.5 * v * y * y)
    return y


def _sc_embed_ln(x_flat, seg_flat, tok_embed, ps_tab, gamma, beta, *, S):
    TOK = x_flat.shape[0]
    EMB = tok_embed.shape[1]
    info = plsc.get_sparse_core_info()
    NC, NS = info.num_cores, info.num_subcores
    NW = NC * NS
    per_w = TOK // NW
    CH = 128
    nch = per_w // CH
    NV = EMB // _LANES
    UNROLL = 4

    mesh = plsc.VectorSubcoreMesh(
        core_axis_name="c", subcore_axis_name="s",
        num_cores=NC, num_subcores=NS)

    def body(x_hbm, seg_hbm, tok_hbm, ps_hbm, gam_hbm, bet_hbm, out_hbm,
             idx_tok, idx_ps, seg_v, tok_buf, ps_buf, gam_v, bet_v,
             sem0, sem1):
        wid = lax.axis_index("s") * NC + lax.axis_index("c")
        pltpu.sync_copy(gam_hbm, gam_v)
        pltpu.sync_copy(bet_hbm, bet_v)
        gs = [gam_v[pl.ds(c * _LANES, _LANES)] for c in range(NV)]
        bs = [bet_v[pl.ds(c * _LANES, _LANES)] for c in range(NV)]

        def chunk(ci, carry):
            row0 = wid * per_w + ci * CH
            pltpu.sync_copy(x_hbm.at[pl.ds(row0, CH)], idx_tok)
            pltpu.sync_copy(seg_hbm.at[pl.ds(row0, CH)], seg_v)
            for j in range(CH // _LANES):
                sv = seg_v[pl.ds(j * _LANES, _LANES)]
                pos = lax.rem(row0 + j * _LANES + lax.iota(jnp.int32, _LANES),
                              jnp.int32(S))
                idx_ps[pl.ds(j * _LANES, _LANES)] = sv * S + pos
            cp0 = pltpu.async_copy(tok_hbm.at[idx_tok], tok_buf, sem0)
            cp1 = pltpu.async_copy(ps_hbm.at[idx_ps], ps_buf, sem1)
            cp0.wait()
            cp1.wait()

            def tok_group(g, carry2):
                for u in range(UNROLL):
                    t = g * UNROLL + u
                    h = [tok_buf[t, pl.ds(c * _LANES, _LANES)]
                         + ps_buf[t, pl.ds(c * _LANES, _LANES)]
                         for c in range(NV)]
                    tot = _lane_sum(sum(h[1:], h[0]))
                    totq = _lane_sum(sum([hc * hc for hc in h[1:]],
                                         h[0] * h[0]))
                    mu = tot * (1.0 / EMB)
                    var = totq * (1.0 / EMB) - mu * mu + _EPS
                    a = _rsqrt(var)
                    for c in range(NV):
                        tok_buf[t, pl.ds(c * _LANES, _LANES)] = (
                            (h[c] - mu) * a * gs[c] + bs[c])
                return carry2

            lax.fori_loop(0, CH // UNROLL, tok_group, 0)
            pltpu.sync_copy(tok_buf, out_hbm.at[pl.ds(row0, CH)])
            return carry

        lax.fori_loop(0, nch, chunk, 0)

    return pl.kernel(
        body,
        out_type=jax.ShapeDtypeStruct((TOK, EMB), jnp.float32),
        mesh=mesh,
        compiler_params=pltpu.CompilerParams(use_tc_tiling_on_sc=False),
        scratch_types=[
            pltpu.VMEM((CH,), jnp.int32),
            pltpu.VMEM((CH,), jnp.int32),
            pltpu.VMEM((CH,), jnp.int32),
            pltpu.VMEM((CH, EMB), jnp.float32),
            pltpu.VMEM((CH, EMB), jnp.float32),
            pltpu.VMEM((EMB,), jnp.float32),
            pltpu.VMEM((EMB,), jnp.float32),
            pltpu.SemaphoreType.DMA,
            pltpu.SemaphoreType.DMA,
        ],
    )(x_flat, seg_flat, tok_embed, ps_tab, gamma, beta)


def kernel(x, seg, tok_embed, pos_embed, seg_embed, gamma, beta):
    B, S = x.shape
    EMB = tok_embed.shape[1]
    ps_tab = _build_ps_table(pos_embed.astype(jnp.float32),
                             seg_embed.astype(jnp.float32), S)
    x_flat = x.reshape(B * S).astype(jnp.int32)
    seg_flat = seg.reshape(B * S).astype(jnp.int32)
    out = _sc_embed_ln(x_flat, seg_flat, tok_embed.astype(jnp.float32),
                       ps_tab, gamma.astype(jnp.float32),
                       beta.astype(jnp.float32), S=S)
    return out.reshape(B, S, EMB)
